# speedup vs baseline: 1.5455x; 1.5455x over previous
import functools

import jax
import jax.numpy as jnp
from jax import lax
from jax.experimental import pallas as pl
from jax.experimental.pallas import tpu as pltpu
from jax.experimental.pallas import tpu_sc as plsc

NUM_WORKERS = 32
CHUNK = 32


def _make_gather(total, embed):
    per_worker = total // NUM_WORKERS
    nchunks = per_worker // CHUNK
    mesh = plsc.VectorSubcoreMesh(core_axis_name="c", subcore_axis_name="s")

    @functools.partial(
        pl.kernel,
        mesh=mesh,
        out_type=jax.ShapeDtypeStruct((total, embed), jnp.float32),
        scratch_types=[
            pltpu.VMEM((per_worker,), jnp.int32),
            pltpu.VMEM((CHUNK, embed), jnp.float32),
            pltpu.VMEM((CHUNK, embed), jnp.float32),
            pltpu.SemaphoreType.DMA,
            pltpu.SemaphoreType.DMA,
            pltpu.SemaphoreType.DMA,
            pltpu.SemaphoreType.DMA,
        ],
    )
    def gather_kernel(idx_hbm, table_hbm, out_hbm, idx_v, buf_a, buf_b,
                      gsem_a, gsem_b, wsem_a, wsem_b):
        wid = lax.axis_index("s") * 2 + lax.axis_index("c")
        base = wid * per_worker
        pltpu.sync_copy(idx_hbm.at[pl.ds(base, per_worker)], idx_v)

        bufs = (buf_a, buf_b)
        gsems = (gsem_a, gsem_b)
        wsems = (wsem_a, wsem_b)

        def start_gather(j):
            return pltpu.async_copy(
                table_hbm.at[idx_v.at[pl.ds(j * CHUNK, CHUNK)]],
                bufs[j % 2], gsems[j % 2])

        gds = [None] * nchunks
        wds = [None] * nchunks
        gds[0] = start_gather(0)
        if nchunks > 1:
            gds[1] = start_gather(1)
        for j in range(nchunks):
            gds[j].wait()
            wds[j] = pltpu.async_copy(
                bufs[j % 2],
                out_hbm.at[pl.ds(base + j * CHUNK, CHUNK)],
                wsems[j % 2])
            if j + 2 < nchunks:
                wds[j].wait()
                gds[j + 2] = start_gather(j + 2)
        if nchunks > 1:
            wds[nchunks - 2].wait()
        wds[nchunks - 1].wait()

    return gather_kernel


def kernel(time, pos_enc):
    shape = time.shape
    idx = time.reshape(-1).astype(jnp.int32)
    total = idx.shape[0]
    out = _make_gather(total, pos_enc.shape[1])(idx, pos_enc)
    return out.reshape(*shape, pos_enc.shape[1])


# speedup vs baseline: 1.5810x; 1.0229x over previous
import functools

import jax
import jax.numpy as jnp
from jax import lax
from jax.experimental import pallas as pl
from jax.experimental.pallas import tpu as pltpu
from jax.experimental.pallas import tpu_sc as plsc

NUM_WORKERS = 32
CHUNK = 32
NBUF = 3


def _make_gather(total, embed):
    per_worker = total // NUM_WORKERS
    nchunks = per_worker // CHUNK
    mesh = plsc.VectorSubcoreMesh(core_axis_name="c", subcore_axis_name="s")

    @functools.partial(
        pl.kernel,
        mesh=mesh,
        out_type=jax.ShapeDtypeStruct((total, embed), jnp.float32),
        scratch_types=[
            pltpu.VMEM((per_worker,), jnp.int32),
        ] + [pltpu.VMEM((CHUNK, embed), jnp.float32)] * NBUF
          + [pltpu.SemaphoreType.DMA] * (2 * NBUF),
    )
    def gather_kernel(idx_hbm, table_hbm, out_hbm, idx_v, *scratch):
        bufs = scratch[:NBUF]
        gsems = scratch[NBUF:2 * NBUF]
        wsems = scratch[2 * NBUF:]
        wid = lax.axis_index("s") * 2 + lax.axis_index("c")
        base = wid * per_worker
        pltpu.sync_copy(idx_hbm.at[pl.ds(base, per_worker)], idx_v)

        def start_gather(j):
            return pltpu.async_copy(
                table_hbm.at[idx_v.at[pl.ds(j * CHUNK, CHUNK)]],
                bufs[j % NBUF], gsems[j % NBUF])

        gds = [None] * nchunks
        wds = [None] * nchunks
        for j in range(min(NBUF, nchunks)):
            gds[j] = start_gather(j)
        for j in range(nchunks):
            gds[j].wait()
            wds[j] = pltpu.async_copy(
                bufs[j % NBUF],
                out_hbm.at[pl.ds(base + j * CHUNK, CHUNK)],
                wsems[j % NBUF])
            nxt = j + NBUF
            if nxt < nchunks:
                wds[j].wait()
                gds[nxt] = start_gather(nxt)
        for j in range(max(0, nchunks - NBUF), nchunks):
            wds[j].wait()

    return gather_kernel


def kernel(time, pos_enc):
    shape = time.shape
    idx = time.reshape(-1).astype(jnp.int32)
    total = idx.shape[0]
    out = _make_gather(total, pos_enc.shape[1])(idx, pos_enc)
    return out.reshape(*shape, pos_enc.shape[1])
